# baseline (speedup 1.0000x reference)
import functools

import jax
import jax.numpy as jnp
from jax import lax
from jax.experimental import pallas as pl
from jax.experimental.pallas import tpu as pltpu
from jax.experimental.pallas import tpu_sc as plsc

D = 100
DP = 128
CHUNK = 128
NSLAB = 4
BB = 64

_info = plsc.get_sparse_core_info()
_NC, _NS = _info.num_cores, _info.num_subcores
NW = _NC * _NS


def _emb_call(n_total, nbuf):
    n_per_w = n_total // NW
    n_chunks = n_per_w // CHUNK
    assert n_chunks % nbuf == 0 and n_chunks >= 2 * nbuf
    mesh = plsc.VectorSubcoreMesh(core_axis_name="c", subcore_axis_name="s")

    @functools.partial(
        pl.kernel,
        out_type=jax.ShapeDtypeStruct((n_total, DP), jnp.float32),
        mesh=mesh,
        scratch_types=[
            pltpu.VMEM((n_chunks, CHUNK), jnp.int32),
            pltpu.VMEM((nbuf, CHUNK, DP), jnp.float32),
        ] + [pltpu.SemaphoreType.DMA] * (2 * nbuf),
        compiler_params=pltpu.CompilerParams(use_tc_tiling_on_sc=False),
    )
    def emb(idx_hbm, table_hbm, out_hbm, idx_v, rows_v, *sems):
        wid = lax.axis_index("s") * _NC + lax.axis_index("c")
        base = wid * n_per_w
        gsems = sems[:nbuf]
        osems = sems[nbuf:]
        pltpu.sync_copy(idx_hbm.at[wid], idx_v)

        def gather(j, slot):
            return pltpu.make_async_copy(
                table_hbm.at[idx_v.at[j]], rows_v.at[slot], gsems[slot])

        def put(j, slot):
            return pltpu.make_async_copy(
                rows_v.at[slot], out_hbm.at[pl.ds(base + j * CHUNK, CHUNK)],
                osems[slot])

        for slot in range(nbuf):
            gather(slot, slot).start()

        def body(i, carry):
            j0 = i * nbuf
            for slot in range(nbuf):
                gather(j0 + slot, slot).wait()
                put(j0 + slot, slot).start()
            for slot in range(nbuf):
                put(j0 + slot, slot).wait()
                gather(j0 + nbuf + slot, slot).start()
            return carry

        lax.fori_loop(0, n_chunks // nbuf - 1, body, 0)

        jlast = n_chunks - nbuf
        for slot in range(nbuf):
            gather(jlast + slot, slot).wait()
            put(jlast + slot, slot).start()
        for slot in range(nbuf):
            put(jlast + slot, slot).wait()

    return emb


def _depad_call(b, s, b_slab, slab, aliased):
    grid = (b_slab // BB,)
    base = slab * (b_slab // BB)

    def body(pad_ref, *rest):
        o_ref = rest[-1]
        o_ref[...] = pad_ref[:, :D].reshape(BB, s, D)

    in_specs = [pl.BlockSpec((BB * s, DP), lambda i: (i, 0))]
    num_inputs = 1
    kwargs = {}
    if aliased:
        in_specs.append(pl.BlockSpec(memory_space=pl.ANY))
        num_inputs = 2
        kwargs["input_output_aliases"] = {1: 0}
    return pl.pallas_call(
        body,
        out_shape=jax.ShapeDtypeStruct((b, s, D), jnp.float32),
        grid=grid,
        in_specs=in_specs,
        out_specs=pl.BlockSpec((BB, s, D), lambda i, base=base: (base + i, 0, 0)),
        **kwargs,
    )


def kernel(x, table):
    b, s = x.shape
    b_slab = b // NSLAB
    n_slab = b_slab * s
    table_p = jnp.pad(table, ((0, 0), (0, DP - D)))
    emb = _emb_call(n_slab, 2)
    padded = []
    for k in range(NSLAB):
        xs = lax.slice_in_dim(x, k * b_slab, (k + 1) * b_slab, axis=0)
        idx = xs.reshape(NW, n_slab // NW // CHUNK, CHUNK).astype(jnp.int32)
        padded.append(emb(idx, table_p))
    out = _depad_call(b, s, b_slab, 0, False)(padded[0])
    for k in range(1, NSLAB):
        out = _depad_call(b, s, b_slab, k, True)(padded[k], out)
    return out


# speedup vs baseline: 1.4156x; 1.4156x over previous
import functools

import jax
import jax.numpy as jnp
from jax import lax
from jax.experimental import pallas as pl
from jax.experimental.pallas import tpu as pltpu
from jax.experimental.pallas import tpu_sc as plsc

D = 100
DP = 128
CHUNK = 128

_info = plsc.get_sparse_core_info()
_NC, _NS = _info.num_cores, _info.num_subcores
NW = _NC * _NS


def _emb_call(n_total):
    n_per_w = n_total // NW
    n_chunks = n_per_w // CHUNK
    NBUF = 5
    assert n_chunks % NBUF == 0
    mesh = plsc.VectorSubcoreMesh(core_axis_name="c", subcore_axis_name="s")

    @functools.partial(
        pl.kernel,
        out_type=jax.ShapeDtypeStruct((n_total, DP), jnp.float32),
        mesh=mesh,
        scratch_types=[
            pltpu.VMEM((n_chunks, CHUNK), jnp.int32),
            pltpu.VMEM((NBUF, CHUNK, DP), jnp.float32),
        ] + [pltpu.SemaphoreType.DMA] * (2 * NBUF),
        compiler_params=pltpu.CompilerParams(use_tc_tiling_on_sc=False),
    )
    def emb(idx_hbm, table_hbm, out_hbm, idx_v, rows_v, *sems):
        wid = lax.axis_index("s") * _NC + lax.axis_index("c")
        base = wid * n_per_w
        gsems = sems[:NBUF]
        osems = sems[NBUF:]
        pltpu.sync_copy(idx_hbm.at[wid], idx_v)

        def gather(j, slot):
            return pltpu.make_async_copy(
                table_hbm.at[idx_v.at[j]], rows_v.at[slot], gsems[slot])

        def put(j, slot):
            return pltpu.make_async_copy(
                rows_v.at[slot], out_hbm.at[pl.ds(base + j * CHUNK, CHUNK)],
                osems[slot])

        for slot in range(NBUF):
            gather(slot, slot).start()

        def body(i, carry):
            j0 = i * NBUF
            for slot in range(NBUF):
                gather(j0 + slot, slot).wait()
                put(j0 + slot, slot).start()
            for slot in range(NBUF):
                put(j0 + slot, slot).wait()
                gather(j0 + NBUF + slot, slot).start()
            return carry

        lax.fori_loop(0, n_chunks // NBUF - 1, body, 0)

        jlast = n_chunks - NBUF
        for slot in range(NBUF):
            gather(jlast + slot, slot).wait()
            put(jlast + slot, slot).start()
        for slot in range(NBUF):
            put(jlast + slot, slot).wait()

    return emb


def kernel(x, table):
    b, s = x.shape
    n_total = b * s
    idx = x.reshape(NW, n_total // NW // CHUNK, CHUNK).astype(jnp.int32)
    table_p = jnp.pad(table, ((0, 0), (0, DP - D)))
    out = _emb_call(n_total)(idx, table_p)
    return out[:, :D].reshape(b, s, D)


# speedup vs baseline: 1.4266x; 1.0078x over previous
import functools

import jax
import jax.numpy as jnp
from jax import lax
from jax.experimental import pallas as pl
from jax.experimental.pallas import tpu as pltpu
from jax.experimental.pallas import tpu_sc as plsc

D = 100
DP = 128
NBUF = 3

_info = plsc.get_sparse_core_info()
_NC, _NS = _info.num_cores, _info.num_subcores
NW = _NC * _NS


def _emb_call(b, s):
    bpw = b // NW
    n_groups = bpw // NBUF
    assert n_groups >= 2
    n_total = b * s
    mesh = plsc.VectorSubcoreMesh(core_axis_name="c", subcore_axis_name="s")

    @functools.partial(
        pl.kernel,
        out_type=jax.ShapeDtypeStruct((n_total, DP), jnp.float32),
        mesh=mesh,
        scratch_types=[
            pltpu.VMEM((bpw, s), jnp.int32),
            pltpu.VMEM((NBUF, s, DP), jnp.float32),
        ] + [pltpu.SemaphoreType.DMA] * (3 * NBUF),
        compiler_params=pltpu.CompilerParams(use_tc_tiling_on_sc=True),
    )
    def emb(x_hbm, table_hbm, out_hbm, xv, rows_v, *sems):
        wid = lax.axis_index("s") * _NC + lax.axis_index("c")
        b0 = wid * bpw
        ga = sems[:NBUF]
        gb = sems[NBUF:2 * NBUF]
        os_ = sems[2 * NBUF:]
        pltpu.sync_copy(x_hbm.at[pl.ds(b0, bpw)], xv)

        def gather_a(r, slot):
            return pltpu.make_async_copy(
                table_hbm.at[xv.at[r, pl.ds(0, 128)]],
                rows_v.at[slot].at[pl.ds(0, 128)], ga[slot])

        def gather_b(r, slot):
            return pltpu.make_async_copy(
                table_hbm.at[xv.at[r, pl.ds(128, s - 128)]],
                rows_v.at[slot].at[pl.ds(128, s - 128)], gb[slot])

        def put(r, slot):
            return pltpu.make_async_copy(
                rows_v.at[slot], out_hbm.at[pl.ds((b0 + r) * s, s)], os_[slot])

        for slot in range(NBUF):
            gather_a(slot, slot).start()
            gather_b(slot, slot).start()

        def body(i, carry):
            r0 = i * NBUF
            for slot in range(NBUF):
                gather_a(r0 + slot, slot).wait()
                gather_b(r0 + slot, slot).wait()
                put(r0 + slot, slot).start()
            for slot in range(NBUF):
                put(r0 + slot, slot).wait()
                gather_a(r0 + NBUF + slot, slot).start()
                gather_b(r0 + NBUF + slot, slot).start()
            return carry

        lax.fori_loop(0, n_groups - 1, body, 0)

        for r in range((n_groups - 1) * NBUF, bpw):
            slot = r % NBUF
            gather_a(r, slot).wait()
            gather_b(r, slot).wait()
            put(r, slot).start()
            nr = r + NBUF
            if nr < bpw:
                put(r, slot).wait()
                gather_a(nr, slot).start()
                gather_b(nr, slot).start()
        for r in range(bpw - NBUF, bpw):
            put(r, r % NBUF).wait()

    return emb


def kernel(x, table):
    b, s = x.shape
    table_p = jnp.pad(table, ((0, 0), (0, DP - D)))
    out = _emb_call(b, s)(x, table_p)
    return out[:, :D].reshape(b, s, D)
